# baseline (speedup 1.0000x reference)
import functools

import jax
import jax.numpy as jnp
from jax import lax
from jax.experimental import pallas as pl
from jax.experimental.pallas import tpu as pltpu
from jax.experimental.pallas import tpu_sc as plsc

_M = 4
_LANES = 16
_NWORKERS = 32
_ROWS = 8
_WIN = _M * _LANES
_NBUF = 4
_AHEAD = 2


def _sc_body(x_hbm, o_hbm, b0_, b1_, b2_, b3_, si0, si1, si2, si3, so0, so1, so2, so3):
    n, d = x_hbm.shape
    chunk = _ROWS * d
    rows_per_w = n // _NWORKERS
    n_chunks = rows_per_w // _ROWS
    bufs = (b0_, b1_, b2_, b3_)
    sis = (si0, si1, si2, si3)
    sos = (so0, so1, so2, so3)
    wid = lax.axis_index("s") * 2 + lax.axis_index("c")
    row0 = wid * rows_per_w
    lane4 = lax.iota(jnp.int32, _LANES) * _M
    zero = jnp.zeros((_LANES,), jnp.float32)

    def load(ci, b):
        r = row0 + ci * _ROWS
        pltpu.make_async_copy(x_hbm.at[pl.ds(r, _ROWS)], bufs[b], sis[b]).start()

    def store(ci, b):
        r = row0 + ci * _ROWS
        pltpu.make_async_copy(bufs[b], o_hbm.at[pl.ds(r, _ROWS)], sos[b]).start()

    def wait_in(b):
        pltpu.make_async_copy(x_hbm.at[pl.ds(row0, _ROWS)], bufs[b], sis[b]).wait()

    def wait_out(b):
        pltpu.make_async_copy(bufs[b], o_hbm.at[pl.ds(row0, _ROWS)], sos[b]).wait()

    def compute(b):
        buf = bufs[b]

        @plsc.parallel_loop(0, chunk, step=_WIN, unroll=4)
        def _(i):
            r = jnp.full((_LANES,), i // d, jnp.int32)
            i0 = lane4 + i % d
            a0 = plsc.load_gather(buf, [r, i0])
            a1 = plsc.load_gather(buf, [r, i0 + 1])
            a2 = plsc.load_gather(buf, [r, i0 + 2])
            a3 = plsc.load_gather(buf, [r, i0 + 3])
            b0 = jnp.abs(a0)
            b1 = jnp.abs(a1)
            b2 = jnp.abs(a2)
            b3 = jnp.abs(a3)
            m1 = jnp.maximum(b0, b1)
            n1 = jnp.minimum(b0, b1)
            m2 = jnp.maximum(b2, b3)
            n2 = jnp.minimum(b2, b3)
            second = jnp.maximum(jnp.minimum(m1, m2), jnp.maximum(n1, n2))
            plsc.store_scatter(buf, [r, i0], jnp.where(b0 >= second, a0, zero))
            plsc.store_scatter(buf, [r, i0 + 1], jnp.where(b1 >= second, a1, zero))
            plsc.store_scatter(buf, [r, i0 + 2], jnp.where(b2 >= second, a2, zero))
            plsc.store_scatter(buf, [r, i0 + 3], jnp.where(b3 >= second, a3, zero))

    for p in range(_NBUF):
        load(p, p)

    def g_body(g, _):
        for b in range(_NBUF):
            ci = g * _NBUF + b
            wait_in(b)
            compute(b)
            store(ci, b)
            p = ci + _AHEAD
            bp = (b + _AHEAD) % _NBUF

            @pl.when((p >= _NBUF) & (p < n_chunks))
            def _():
                wait_out(bp)
                load(p, bp)

        return 0

    lax.fori_loop(0, n_chunks // _NBUF, g_body, 0)
    for b in range(_NBUF):
        wait_out(b)


def kernel(input):
    n, d = input.shape
    assert n % (_NWORKERS * _ROWS * _NBUF) == 0 and d % _WIN == 0
    mesh = plsc.VectorSubcoreMesh(core_axis_name="c", subcore_axis_name="s")
    return pl.kernel(
        _sc_body,
        out_type=jax.ShapeDtypeStruct((n, d), jnp.float32),
        mesh=mesh,
        scratch_types=[pltpu.VMEM((_ROWS, d), jnp.float32)] * _NBUF
        + [pltpu.SemaphoreType.DMA] * (2 * _NBUF),
        compiler_params=pltpu.CompilerParams(
            needs_layout_passes=False, use_tc_tiling_on_sc=True
        ),
    )(input)
